# Initial kernel scaffold; baseline (speedup 1.0000x reference)
import jax
import jax.numpy as jnp
from jax.experimental import pallas as pl


def kernel(nfeats, efeats, edge_index, We0, be0, Wn0, bn0, We1, be1, Wn1, bn1, Wmn1, bmn1, Wmn2, bmn2, Wme1, bme1, Wme2, bme2):
    raise NotImplementedError("write your pallas kernel here")


# speedup vs baseline: 3.5847x; 3.5847x over previous
import functools

import jax
import jax.numpy as jnp
from jax import lax
from jax.experimental import pallas as pl
from jax.experimental.pallas import tpu as pltpu
from jax.experimental.pallas import tpu_sc as plsc

N = 10000
E = 320000
DN = 128
DE = 16

NC = 2
NS = 16
NW = NC * NS
CHUNK = 128
K = 79
TPE = K * CHUNK
E_PAD = NW * TPE
N_PAD = N + 16
NB = 4
RN = N_PAD // NB
RB = 512
GE = E_PAD // 8 // RB
EPS = 1e-5

_HIGH = jax.lax.Precision.HIGHEST


def _sc_gather_body(ps_h, pd_h, q_h, src_h, dst_h, x_h, sums_h,
                    src_v, dst_v, gs_v, gd_v, q_v, sums_v, sem1, sem2):
    cid = lax.axis_index("c")
    sid = lax.axis_index("s")
    wid = sid * NC + cid
    pltpu.sync_copy(src_h.at[wid], src_v)
    pltpu.sync_copy(dst_h.at[wid], dst_v)
    base = wid * TPE

    def chunk(j, carry):
        s, s2 = carry
        cp1 = pltpu.async_copy(ps_h.at[src_v.at[j]], gs_v, sem1)
        cp2 = pltpu.async_copy(pd_h.at[dst_v.at[j]], gd_v, sem2)
        pltpu.sync_copy(q_h.at[pl.ds(base + j * CHUNK, CHUNK)], q_v)
        cp1.wait()
        cp2.wait()

        def row(i, c):
            a, a2 = c
            x = gs_v[i] + gd_v[i] + q_v[i]
            q_v[i] = x
            return (a + x, a2 + x * x)

        s, s2 = lax.fori_loop(0, CHUNK, row, (s, s2))
        pltpu.sync_copy(q_v, x_h.at[pl.ds(base + j * CHUNK, CHUNK)])
        return (s, s2)

    z = jnp.zeros((DE,), jnp.float32)
    s, s2 = lax.fori_loop(0, K, chunk, (z, z))
    sums_v[0] = s
    sums_v[1] = s2
    pltpu.sync_copy(sums_v, sums_h.at[wid])


def _sc_scatter_body(e_h, dst_h, z_h, agg_h, dst_v, e_v, acc):
    cid = lax.axis_index("c")
    sid = lax.axis_index("s")
    wid = sid * NC + cid

    @pl.when(sid == 0)
    def _():
        pltpu.sync_copy(z_h, acc)

    plsc.subcore_barrier()
    pltpu.sync_copy(dst_h.at[wid], dst_v)
    base = wid * TPE

    def chunk(j, _):
        pltpu.sync_copy(e_h.at[pl.ds(base + j * CHUNK, CHUNK)], e_v)
        pltpu.sync_copy(e_v, acc.at[dst_v.at[j]], add=True)
        return 0

    lax.fori_loop(0, K, chunk, 0)
    plsc.subcore_barrier()
    rp = N_PAD // NS
    pltpu.sync_copy(acc.at[pl.ds(sid * rp, rp)],
                    agg_h.at[cid, pl.ds(sid * rp, rp)])


@functools.cache
def _sc_kernels():
    mesh = plsc.VectorSubcoreMesh(core_axis_name="c", subcore_axis_name="s")
    params = pltpu.CompilerParams(use_tc_tiling_on_sc=False)
    gather = pl.kernel(
        _sc_gather_body,
        compiler_params=params,
        out_type=(
            jax.ShapeDtypeStruct((E_PAD, DE), jnp.float32),
            jax.ShapeDtypeStruct((NW, 2, DE), jnp.float32),
        ),
        mesh=mesh,
        scratch_types=[
            pltpu.VMEM((K, CHUNK), jnp.int32),
            pltpu.VMEM((K, CHUNK), jnp.int32),
            pltpu.VMEM((CHUNK, DE), jnp.float32),
            pltpu.VMEM((CHUNK, DE), jnp.float32),
            pltpu.VMEM((CHUNK, DE), jnp.float32),
            pltpu.VMEM((2, DE), jnp.float32),
            pltpu.SemaphoreType.DMA,
            pltpu.SemaphoreType.DMA,
        ],
    )
    scatter = pl.kernel(
        _sc_scatter_body,
        compiler_params=params,
        out_type=jax.ShapeDtypeStruct((NC, N_PAD, DE), jnp.float32),
        mesh=mesh,
        scratch_types=[
            pltpu.VMEM((K, CHUNK), jnp.int32),
            pltpu.VMEM((CHUNK, DE), jnp.float32),
            pltpu.VMEM_SHARED((N_PAD, DE), jnp.float32),
        ],
    )
    return gather, scatter


def _sc_gather(ps, pd, q, src_r, dst_r):
    return _sc_kernels()[0](ps, pd, q, src_r, dst_r)


def _sc_scatter(e, dst_r, zeros_n):
    return _sc_kernels()[1](e, dst_r, zeros_n)


def _dot(a, b):
    return jnp.dot(a, b, preferred_element_type=jnp.float32, precision=_HIGH)


def _tc_ptables_body(nf_ref, w_ref, ps_ref, pd_ref):
    p = _dot(nf_ref[...], w_ref[...])
    ps_ref[...] = p[:, :DE]
    pd_ref[...] = p[:, DE:]


def _tc_ptables(nf_p, w_sd):
    return pl.pallas_call(
        _tc_ptables_body,
        grid=(NB,),
        in_specs=[
            pl.BlockSpec((RN, DN), lambda g: (g, 0)),
            pl.BlockSpec((DN, 32), lambda g: (0, 0)),
        ],
        out_specs=(
            pl.BlockSpec((RN, DE), lambda g: (g, 0)),
            pl.BlockSpec((RN, DE), lambda g: (g, 0)),
        ),
        out_shape=(
            jax.ShapeDtypeStruct((N_PAD, DE), jnp.float32),
            jax.ShapeDtypeStruct((N_PAD, DE), jnp.float32),
        ),
    )(nf_p, w_sd)


def _tc_edge_pre_body(ef_ref, w_ref, q_ref):
    q_ref[...] = _dot(ef_ref[...], w_ref[...])


def _tc_edge_pre(ef_r, w_blk):
    return pl.pallas_call(
        _tc_edge_pre_body,
        grid=(GE,),
        in_specs=[
            pl.BlockSpec((RB, 128), lambda g: (g, 0)),
            pl.BlockSpec((128, 128), lambda g: (0, 0)),
        ],
        out_specs=pl.BlockSpec((RB, 128), lambda g: (g, 0)),
        out_shape=jax.ShapeDtypeStruct((E_PAD // 8, 128), jnp.float32),
    )(ef_r, w_blk)


def _bn_sin_block(x, sums, g):
    t = jnp.sum(sums, axis=0)
    mu = t[:DE] / E
    var = t[DE:] / E - mu * mu
    inv = lax.rsqrt(var + EPS)
    mu128 = jnp.tile(mu, 8)
    inv128 = jnp.tile(inv, 8)
    rid = g * RB + lax.broadcasted_iota(jnp.int32, (RB, 1), 0)
    valid = rid < (E // 8)
    return jnp.where(valid, jnp.sin((x - mu128[None, :]) * inv128[None, :]), 0.0)


def _tc_edge_act0_body(x_ref, sums_ref, w_ref, e_ref, q_ref):
    g = pl.program_id(0)
    e = _bn_sin_block(x_ref[...], sums_ref[...], g)
    e_ref[...] = e
    q_ref[...] = _dot(e, w_ref[...])


def _tc_edge_act0(x_r, sums, w_blk_next):
    return pl.pallas_call(
        _tc_edge_act0_body,
        grid=(GE,),
        in_specs=[
            pl.BlockSpec((RB, 128), lambda g: (g, 0)),
            pl.BlockSpec((NW, 32), lambda g: (0, 0)),
            pl.BlockSpec((128, 128), lambda g: (0, 0)),
        ],
        out_specs=(
            pl.BlockSpec((RB, 128), lambda g: (g, 0)),
            pl.BlockSpec((RB, 128), lambda g: (g, 0)),
        ),
        out_shape=(
            jax.ShapeDtypeStruct((E_PAD // 8, 128), jnp.float32),
            jax.ShapeDtypeStruct((E_PAD // 8, 128), jnp.float32),
        ),
    )(x_r, sums, w_blk_next)


def _tc_edge_act1_body(x_ref, sums_ref, w1_ref, b1_ref, w2_ref, b2_ref,
                       e_ref, ef_ref):
    g = pl.program_id(0)
    e = _bn_sin_block(x_ref[...], sums_ref[...], g)
    e_ref[...] = e
    h = jnp.sin(_dot(e, w1_ref[...]) + b1_ref[...])
    ef_ref[...] = _dot(h, w2_ref[...]) + b2_ref[...]


def _tc_edge_act1(x_r, sums, w1k, b1t, w2k, b2t):
    return pl.pallas_call(
        _tc_edge_act1_body,
        grid=(GE,),
        in_specs=[
            pl.BlockSpec((RB, 128), lambda g: (g, 0)),
            pl.BlockSpec((NW, 32), lambda g: (0, 0)),
            pl.BlockSpec((128, 256), lambda g: (0, 0)),
            pl.BlockSpec((1, 256), lambda g: (0, 0)),
            pl.BlockSpec((256, 128), lambda g: (0, 0)),
            pl.BlockSpec((1, 128), lambda g: (0, 0)),
        ],
        out_specs=(
            pl.BlockSpec((RB, 128), lambda g: (g, 0)),
            pl.BlockSpec((RB, 128), lambda g: (g, 0)),
        ),
        out_shape=(
            jax.ShapeDtypeStruct((E_PAD // 8, 128), jnp.float32),
            jax.ShapeDtypeStruct((E_PAD // 8, 128), jnp.float32),
        ),
    )(x_r, sums, w1k, b1t, w2k, b2t)


def _tc_node_mm_body(nf_ref, agg_ref, wnn_ref, wna_ref, y_ref, sums_ref):
    g = pl.program_id(0)
    agg = agg_ref[0] + agg_ref[1]
    y = _dot(nf_ref[...], wnn_ref[...]) + _dot(agg, wna_ref[...])
    rid = g * RN + lax.broadcasted_iota(jnp.int32, (RN, 1), 0)
    y = jnp.where(rid < N, y, 0.0)
    y_ref[...] = y
    sums_ref[0, 0] = jnp.sum(y, axis=0)
    sums_ref[0, 1] = jnp.sum(y * y, axis=0)


def _tc_node_mm(nf_p, agg, wnn, wna):
    return pl.pallas_call(
        _tc_node_mm_body,
        grid=(NB,),
        in_specs=[
            pl.BlockSpec((RN, DN), lambda g: (g, 0)),
            pl.BlockSpec((NC, RN, DE), lambda g: (0, g, 0)),
            pl.BlockSpec((DN, DN), lambda g: (0, 0)),
            pl.BlockSpec((DE, DN), lambda g: (0, 0)),
        ],
        out_specs=(
            pl.BlockSpec((RN, DN), lambda g: (g, 0)),
            pl.BlockSpec((1, 2, DN), lambda g: (g, 0, 0)),
        ),
        out_shape=(
            jax.ShapeDtypeStruct((N_PAD, DN), jnp.float32),
            jax.ShapeDtypeStruct((NB, 2, DN), jnp.float32),
        ),
    )(nf_p, agg, wnn, wna)


def _node_bn_block(y, sums, g):
    t = jnp.sum(sums, axis=0)
    mu = t[0] / N
    var = t[1] / N - mu * mu
    inv = lax.rsqrt(var + EPS)
    act = jnp.sin((y - mu[None, :]) * inv[None, :])
    rid = g * RN + lax.broadcasted_iota(jnp.int32, (RN, 1), 0)
    return jnp.where(rid < N, act, 0.0)


def _tc_node_apply_body(y_ref, sums_ref, wsd_ref, nf1_ref, ps_ref, pd_ref):
    nf1 = _node_bn_block(y_ref[...], sums_ref[...], pl.program_id(0))
    nf1_ref[...] = nf1
    p = _dot(nf1, wsd_ref[...])
    ps_ref[...] = p[:, :DE]
    pd_ref[...] = p[:, DE:]


def _tc_node_apply(y, sums, wsd_next):
    return pl.pallas_call(
        _tc_node_apply_body,
        grid=(NB,),
        in_specs=[
            pl.BlockSpec((RN, DN), lambda g: (g, 0)),
            pl.BlockSpec((NB, 2, DN), lambda g: (0, 0, 0)),
            pl.BlockSpec((DN, 32), lambda g: (0, 0)),
        ],
        out_specs=(
            pl.BlockSpec((RN, DN), lambda g: (g, 0)),
            pl.BlockSpec((RN, DE), lambda g: (g, 0)),
            pl.BlockSpec((RN, DE), lambda g: (g, 0)),
        ),
        out_shape=(
            jax.ShapeDtypeStruct((N_PAD, DN), jnp.float32),
            jax.ShapeDtypeStruct((N_PAD, DE), jnp.float32),
            jax.ShapeDtypeStruct((N_PAD, DE), jnp.float32),
        ),
    )(y, sums, wsd_next)


def _tc_node_final_body(y_ref, sums_ref, w1_ref, b1_ref, w2_ref, b2_ref,
                        out_ref):
    nf2 = _node_bn_block(y_ref[...], sums_ref[...], pl.program_id(0))
    h = jnp.sin(_dot(nf2, w1_ref[...]) + b1_ref[...])
    out_ref[...] = _dot(h, w2_ref[...]) + b2_ref[...]


def _tc_node_final(y, sums, w1, b1, w2, b2):
    return pl.pallas_call(
        _tc_node_final_body,
        grid=(NB,),
        in_specs=[
            pl.BlockSpec((RN, DN), lambda g: (g, 0)),
            pl.BlockSpec((NB, 2, DN), lambda g: (0, 0, 0)),
            pl.BlockSpec((DN, 256), lambda g: (0, 0)),
            pl.BlockSpec((1, 256), lambda g: (0, 0)),
            pl.BlockSpec((256, DN), lambda g: (0, 0)),
            pl.BlockSpec((1, DN), lambda g: (0, 0)),
        ],
        out_specs=pl.BlockSpec((RN, DN), lambda g: (g, 0)),
        out_shape=jax.ShapeDtypeStruct((N_PAD, DN), jnp.float32),
    )(y, sums, w1, b1, w2, b2)


def kernel(nfeats, efeats, edge_index, We0, be0, Wn0, bn0, We1, be1, Wn1, bn1,
           Wmn1, bmn1, Wmn2, bmn2, Wme1, bme1, Wme2, bme2):
    pad = E_PAD - E
    src_r = jnp.concatenate(
        [edge_index[0], jnp.full((pad,), N, jnp.int32)]).reshape(NW, K, CHUNK)
    dst_r = jnp.concatenate(
        [edge_index[1], jnp.full((pad,), N, jnp.int32)]).reshape(NW, K, CHUNK)
    ef_r = jnp.concatenate(
        [efeats, jnp.zeros((pad, DE), jnp.float32)]).reshape(E_PAD // 8, 128)
    zeros_n = jnp.zeros((N_PAD, DE), jnp.float32)

    i8 = jnp.eye(8, dtype=jnp.float32)
    w_blk0 = jnp.kron(i8, We0[2 * DN:])
    w_blk1 = jnp.kron(i8, We1[2 * DN:])
    w_me1k = jnp.kron(i8, Wme1)
    w_me2k = jnp.kron(i8, Wme2)
    b_me1t = jnp.tile(bme1, 8).reshape(1, 256)
    b_me2t = jnp.tile(bme2, 8).reshape(1, 128)
    w_sd0 = jnp.concatenate([We0[:DN], We0[DN:2 * DN]], axis=1)
    w_sd1 = jnp.concatenate([We1[:DN], We1[DN:2 * DN]], axis=1)
    nf_p = jnp.concatenate(
        [nfeats, jnp.zeros((N_PAD - N, DN), jnp.float32)], axis=0)

    ps0, pd0 = _tc_ptables(nf_p, w_sd0)
    q0_r = _tc_edge_pre(ef_r, w_blk0)
    x0, sums0 = _sc_gather(ps0, pd0, q0_r.reshape(E_PAD, DE), src_r, dst_r)
    e0_r, q1_r = _tc_edge_act0(x0.reshape(E_PAD // 8, 128),
                               sums0.reshape(NW, 32), w_blk1)
    agg0 = _sc_scatter(e0_r.reshape(E_PAD, DE), dst_r, zeros_n)
    y0, ns0 = _tc_node_mm(nf_p, agg0, Wn0[:DN], Wn0[DN:])
    nf1_p, ps1, pd1 = _tc_node_apply(y0, ns0, w_sd1)

    x1, sums1 = _sc_gather(ps1, pd1, q1_r.reshape(E_PAD, DE), src_r, dst_r)
    e1_r, ef_out_r = _tc_edge_act1(x1.reshape(E_PAD // 8, 128),
                                   sums1.reshape(NW, 32),
                                   w_me1k, b_me1t, w_me2k, b_me2t)
    agg1 = _sc_scatter(e1_r.reshape(E_PAD, DE), dst_r, zeros_n)

    y1, ns1 = _tc_node_mm(nf1_p, agg1, Wn1[:DN], Wn1[DN:])
    nf_out_p = _tc_node_final(y1, ns1, Wmn1, bmn1.reshape(1, -1),
                              Wmn2, bmn2.reshape(1, -1))
    ef_out = ef_out_r.reshape(E_PAD, DE)[:E]
    return nf_out_p[:N], ef_out


# speedup vs baseline: 4.5731x; 1.2757x over previous
import functools

import jax
import jax.numpy as jnp
from jax import lax
from jax.experimental import pallas as pl
from jax.experimental.pallas import tpu as pltpu
from jax.experimental.pallas import tpu_sc as plsc

N = 10000
E = 320000
DN = 128
DE = 16

NC = 2
NS = 16
NW = NC * NS
CHUNK = 128
K = 80
TPE = K * CHUNK
I_PAD = NW * TPE
EV = E // 8
CV = CHUNK // 8
N_PAD = N + 16
NB = 4
RN = N_PAD // NB
RB = 400
GE = EV // RB
EPS = 1e-5

_HIGH = jax.lax.Precision.HIGHEST


def _sc_gather_body(ps_h, pd_h, q_h, src_h, dst_h, x_h, sums_h,
                    src_v, dst_v, gs0, gs1, gd0, gd1, qx0, qx1, xv0, xv1,
                    sums_v, ss0, ss1, sd0, sd1, sq0, sq1, so0, so1):
    cid = lax.axis_index("c")
    sid = lax.axis_index("s")
    wid = sid * NC + cid
    pltpu.sync_copy(src_h.at[wid], src_v)
    pltpu.sync_copy(dst_h.at[wid], dst_v)
    vrow0 = wid * (TPE // 8)
    vc = jnp.minimum(E - wid * TPE, TPE) // CHUNK
    bufs = ((gs0, gd0, qx0, xv0, ss0, sd0, sq0, so0),
            (gs1, gd1, qx1, xv1, ss1, sd1, sq1, so1))

    def fire(j, b):
        gs, gd, qx = bufs[b][0], bufs[b][1], bufs[b][2]
        ss, sd, sq = bufs[b][4], bufs[b][5], bufs[b][6]
        pltpu.async_copy(ps_h.at[src_v.at[j]], gs, ss)
        pltpu.async_copy(pd_h.at[dst_v.at[j]], gd, sd)
        pltpu.async_copy(q_h.at[pl.ds(vrow0 + j * CV, CV)], qx, sq)

    def wait_in(j, b):
        gs, gd, qx = bufs[b][0], bufs[b][1], bufs[b][2]
        ss, sd, sq = bufs[b][4], bufs[b][5], bufs[b][6]
        pltpu.make_async_copy(ps_h.at[src_v.at[j]], gs, ss).wait()
        pltpu.make_async_copy(pd_h.at[dst_v.at[j]], gd, sd).wait()
        pltpu.make_async_copy(q_h.at[pl.ds(vrow0 + j * CV, CV)], qx, sq).wait()

    fire(0, 0)
    fire(1, 1)
    zv = jnp.zeros((DE,), jnp.float32)
    carry0 = (zv,) * 16

    def pair(t, carry):
        for b in (0, 1):
            gs, gd, qx, xv, _, _, _, so = bufs[b]
            j = 2 * t + b
            wait_in(j, b)

            @pl.when(t > 0)
            def _():
                pltpu.make_async_copy(
                    xv, x_h.at[pl.ds(vrow0 + (j - 2) * CV, CV)], so).wait()

            def rowblk(r, c):
                c = list(c)
                for m in range(8):
                    i = 8 * r + m
                    x = gs[i] + gd[i] + qx[r, pl.ds(16 * m, 16)]
                    xv[r, pl.ds(16 * m, 16)] = x
                    c[m] = c[m] + x
                    c[8 + m] = c[8 + m] + x * x
                return tuple(c)

            carry = lax.fori_loop(0, CV, rowblk, carry)
            pltpu.async_copy(xv, x_h.at[pl.ds(vrow0 + j * CV, CV)], so)

            @pl.when(j + 2 < vc)
            def _():
                fire(j + 2, b)
        return carry

    carry = lax.fori_loop(0, vc // 2, pair, carry0)
    for b in (0, 1):
        xv, so = bufs[b][3], bufs[b][7]
        j_last = vc - 2 + b
        pltpu.make_async_copy(
            xv, x_h.at[pl.ds(vrow0 + j_last * CV, CV)], so).wait()
    s = carry[0]
    s2 = carry[8]
    for m in range(1, 8):
        s = s + carry[m]
        s2 = s2 + carry[8 + m]
    sums_v[0] = s
    sums_v[1] = s2
    pltpu.sync_copy(sums_v, sums_h.at[wid])


def _sc_scatter_body(e_h, dst_h, z_h, agg_h, dst_v, ev0, ev1, et0, et1, acc,
                     sl0, sl1, sw0, sw1):
    cid = lax.axis_index("c")
    sid = lax.axis_index("s")
    wid = sid * NC + cid

    @pl.when(sid == 0)
    def _():
        pltpu.sync_copy(z_h, acc)

    plsc.subcore_barrier()
    pltpu.sync_copy(dst_h.at[wid], dst_v)
    vrow0 = wid * (TPE // 8)
    vc = jnp.minimum(E - wid * TPE, TPE) // CHUNK
    bufs = ((ev0, et0, sl0, sw0), (ev1, et1, sl1, sw1))

    def fire(j, b):
        ev, _, sl, _ = bufs[b]
        pltpu.async_copy(e_h.at[pl.ds(vrow0 + j * CV, CV)], ev, sl)

    fire(0, 0)
    fire(1, 1)

    def pair(t, _):
        for b in (0, 1):
            ev, et, sl, sw = bufs[b]
            j = 2 * t + b
            pltpu.make_async_copy(
                e_h.at[pl.ds(vrow0 + j * CV, CV)], ev, sl).wait()

            @pl.when(t > 0)
            def _():
                pltpu.make_async_copy(et, acc.at[dst_v.at[j - 2]], sw).wait()

            def rowblk(r, c):
                for m in range(8):
                    et[8 * r + m] = ev[r, pl.ds(16 * m, 16)]
                return c

            lax.fori_loop(0, CV, rowblk, 0)
            pltpu.async_copy(et, acc.at[dst_v.at[j]], sw, add=True)

            @pl.when(j + 2 < vc)
            def _():
                fire(j + 2, b)
        return 0

    lax.fori_loop(0, vc // 2, pair, 0)
    for b in (0, 1):
        _, et, _, sw = bufs[b]
        j_last = vc - 2 + b
        pltpu.make_async_copy(et, acc.at[dst_v.at[j_last]], sw).wait()
    plsc.subcore_barrier()
    rp = N_PAD // NS
    pltpu.sync_copy(acc.at[pl.ds(sid * rp, rp)],
                    agg_h.at[cid, pl.ds(sid * rp, rp)])


@functools.cache
def _sc_kernels():
    mesh = plsc.VectorSubcoreMesh(core_axis_name="c", subcore_axis_name="s")
    params = pltpu.CompilerParams(use_tc_tiling_on_sc=False)
    gather = pl.kernel(
        _sc_gather_body,
        compiler_params=params,
        out_type=(
            jax.ShapeDtypeStruct((EV, 128), jnp.float32),
            jax.ShapeDtypeStruct((NW, 2, DE), jnp.float32),
        ),
        mesh=mesh,
        scratch_types=[
            pltpu.VMEM((K, CHUNK), jnp.int32),
            pltpu.VMEM((K, CHUNK), jnp.int32),
            pltpu.VMEM((CHUNK, DE), jnp.float32),
            pltpu.VMEM((CHUNK, DE), jnp.float32),
            pltpu.VMEM((CHUNK, DE), jnp.float32),
            pltpu.VMEM((CHUNK, DE), jnp.float32),
            pltpu.VMEM((CV, 128), jnp.float32),
            pltpu.VMEM((CV, 128), jnp.float32),
            pltpu.VMEM((CV, 128), jnp.float32),
            pltpu.VMEM((CV, 128), jnp.float32),
            pltpu.VMEM((2, DE), jnp.float32),
        ] + [pltpu.SemaphoreType.DMA] * 8,
    )
    scatter = pl.kernel(
        _sc_scatter_body,
        compiler_params=params,
        out_type=jax.ShapeDtypeStruct((NC, N_PAD, DE), jnp.float32),
        mesh=mesh,
        scratch_types=[
            pltpu.VMEM((K, CHUNK), jnp.int32),
            pltpu.VMEM((CV, 128), jnp.float32),
            pltpu.VMEM((CV, 128), jnp.float32),
            pltpu.VMEM((CHUNK, DE), jnp.float32),
            pltpu.VMEM((CHUNK, DE), jnp.float32),
            pltpu.VMEM_SHARED((N_PAD, DE), jnp.float32),
        ] + [pltpu.SemaphoreType.DMA] * 4,
    )
    return gather, scatter


def _sc_gather(ps, pd, q, src_r, dst_r):
    return _sc_kernels()[0](ps, pd, q, src_r, dst_r)


def _sc_scatter(e, dst_r, zeros_n):
    return _sc_kernels()[1](e, dst_r, zeros_n)


def _dot(a, b):
    return jnp.dot(a, b, preferred_element_type=jnp.float32, precision=_HIGH)


def _tc_ptables_body(nf_ref, w_ref, ps_ref, pd_ref):
    p = _dot(nf_ref[...], w_ref[...])
    ps_ref[...] = p[:, :DE]
    pd_ref[...] = p[:, DE:]


def _tc_ptables(nf_p, w_sd):
    return pl.pallas_call(
        _tc_ptables_body,
        grid=(NB,),
        in_specs=[
            pl.BlockSpec((RN, DN), lambda g: (g, 0)),
            pl.BlockSpec((DN, 32), lambda g: (0, 0)),
        ],
        out_specs=(
            pl.BlockSpec((RN, DE), lambda g: (g, 0)),
            pl.BlockSpec((RN, DE), lambda g: (g, 0)),
        ),
        out_shape=(
            jax.ShapeDtypeStruct((N_PAD, DE), jnp.float32),
            jax.ShapeDtypeStruct((N_PAD, DE), jnp.float32),
        ),
    )(nf_p, w_sd)


def _tc_edge_pre_body(ef_ref, w_ref, q_ref):
    q_ref[...] = _dot(ef_ref[...], w_ref[...])


def _tc_edge_pre(ef_r, w_blk):
    return pl.pallas_call(
        _tc_edge_pre_body,
        grid=(GE,),
        in_specs=[
            pl.BlockSpec((RB, 128), lambda g: (g, 0)),
            pl.BlockSpec((128, 128), lambda g: (0, 0)),
        ],
        out_specs=pl.BlockSpec((RB, 128), lambda g: (g, 0)),
        out_shape=jax.ShapeDtypeStruct((EV, 128), jnp.float32),
    )(ef_r, w_blk)


def _bn_sin_block(x, sums):
    t = jnp.sum(sums, axis=0)
    mu = t[:DE] / E
    var = t[DE:] / E - mu * mu
    inv = lax.rsqrt(var + EPS)
    mu128 = jnp.tile(mu, 8)
    inv128 = jnp.tile(inv, 8)
    return jnp.sin((x - mu128[None, :]) * inv128[None, :])


def _tc_edge_act0_body(x_ref, sums_ref, w_ref, e_ref, q_ref):
    e = _bn_sin_block(x_ref[...], sums_ref[...])
    e_ref[...] = e
    q_ref[...] = _dot(e, w_ref[...])


def _tc_edge_act0(x_r, sums, w_blk_next):
    return pl.pallas_call(
        _tc_edge_act0_body,
        grid=(GE,),
        in_specs=[
            pl.BlockSpec((RB, 128), lambda g: (g, 0)),
            pl.BlockSpec((NW, 32), lambda g: (0, 0)),
            pl.BlockSpec((128, 128), lambda g: (0, 0)),
        ],
        out_specs=(
            pl.BlockSpec((RB, 128), lambda g: (g, 0)),
            pl.BlockSpec((RB, 128), lambda g: (g, 0)),
        ),
        out_shape=(
            jax.ShapeDtypeStruct((EV, 128), jnp.float32),
            jax.ShapeDtypeStruct((EV, 128), jnp.float32),
        ),
    )(x_r, sums, w_blk_next)


def _tc_edge_act1_body(x_ref, sums_ref, w1_ref, b1_ref, w2_ref, b2_ref,
                       e_ref, ef_ref):
    e = _bn_sin_block(x_ref[...], sums_ref[...])
    e_ref[...] = e
    h = jnp.sin(_dot(e, w1_ref[...]) + b1_ref[...])
    ef_ref[...] = _dot(h, w2_ref[...]) + b2_ref[...]


def _tc_edge_act1(x_r, sums, w1k, b1t, w2k, b2t):
    return pl.pallas_call(
        _tc_edge_act1_body,
        grid=(GE,),
        in_specs=[
            pl.BlockSpec((RB, 128), lambda g: (g, 0)),
            pl.BlockSpec((NW, 32), lambda g: (0, 0)),
            pl.BlockSpec((128, 256), lambda g: (0, 0)),
            pl.BlockSpec((1, 256), lambda g: (0, 0)),
            pl.BlockSpec((256, 128), lambda g: (0, 0)),
            pl.BlockSpec((1, 128), lambda g: (0, 0)),
        ],
        out_specs=(
            pl.BlockSpec((RB, 128), lambda g: (g, 0)),
            pl.BlockSpec((RB, 128), lambda g: (g, 0)),
        ),
        out_shape=(
            jax.ShapeDtypeStruct((EV, 128), jnp.float32),
            jax.ShapeDtypeStruct((EV, 128), jnp.float32),
        ),
    )(x_r, sums, w1k, b1t, w2k, b2t)


def _tc_node_mm_body(nf_ref, agg_ref, wnn_ref, wna_ref, y_ref, sums_ref):
    g = pl.program_id(0)
    agg = agg_ref[0] + agg_ref[1]
    y = _dot(nf_ref[...], wnn_ref[...]) + _dot(agg, wna_ref[...])
    rid = g * RN + lax.broadcasted_iota(jnp.int32, (RN, 1), 0)
    y = jnp.where(rid < N, y, 0.0)
    y_ref[...] = y
    sums_ref[0, 0] = jnp.sum(y, axis=0)
    sums_ref[0, 1] = jnp.sum(y * y, axis=0)


def _tc_node_mm(nf_p, agg, wnn, wna):
    return pl.pallas_call(
        _tc_node_mm_body,
        grid=(NB,),
        in_specs=[
            pl.BlockSpec((RN, DN), lambda g: (g, 0)),
            pl.BlockSpec((NC, RN, DE), lambda g: (0, g, 0)),
            pl.BlockSpec((DN, DN), lambda g: (0, 0)),
            pl.BlockSpec((DE, DN), lambda g: (0, 0)),
        ],
        out_specs=(
            pl.BlockSpec((RN, DN), lambda g: (g, 0)),
            pl.BlockSpec((1, 2, DN), lambda g: (g, 0, 0)),
        ),
        out_shape=(
            jax.ShapeDtypeStruct((N_PAD, DN), jnp.float32),
            jax.ShapeDtypeStruct((NB, 2, DN), jnp.float32),
        ),
    )(nf_p, agg, wnn, wna)


def _node_bn_block(y, sums, g):
    t = jnp.sum(sums, axis=0)
    mu = t[0] / N
    var = t[1] / N - mu * mu
    inv = lax.rsqrt(var + EPS)
    act = jnp.sin((y - mu[None, :]) * inv[None, :])
    rid = g * RN + lax.broadcasted_iota(jnp.int32, (RN, 1), 0)
    return jnp.where(rid < N, act, 0.0)


def _tc_node_apply_body(y_ref, sums_ref, wsd_ref, nf1_ref, ps_ref, pd_ref):
    nf1 = _node_bn_block(y_ref[...], sums_ref[...], pl.program_id(0))
    nf1_ref[...] = nf1
    p = _dot(nf1, wsd_ref[...])
    ps_ref[...] = p[:, :DE]
    pd_ref[...] = p[:, DE:]


def _tc_node_apply(y, sums, wsd_next):
    return pl.pallas_call(
        _tc_node_apply_body,
        grid=(NB,),
        in_specs=[
            pl.BlockSpec((RN, DN), lambda g: (g, 0)),
            pl.BlockSpec((NB, 2, DN), lambda g: (0, 0, 0)),
            pl.BlockSpec((DN, 32), lambda g: (0, 0)),
        ],
        out_specs=(
            pl.BlockSpec((RN, DN), lambda g: (g, 0)),
            pl.BlockSpec((RN, DE), lambda g: (g, 0)),
            pl.BlockSpec((RN, DE), lambda g: (g, 0)),
        ),
        out_shape=(
            jax.ShapeDtypeStruct((N_PAD, DN), jnp.float32),
            jax.ShapeDtypeStruct((N_PAD, DE), jnp.float32),
            jax.ShapeDtypeStruct((N_PAD, DE), jnp.float32),
        ),
    )(y, sums, wsd_next)


def _tc_node_final_body(y_ref, sums_ref, w1_ref, b1_ref, w2_ref, b2_ref,
                        out_ref):
    nf2 = _node_bn_block(y_ref[...], sums_ref[...], pl.program_id(0))
    h = jnp.sin(_dot(nf2, w1_ref[...]) + b1_ref[...])
    out_ref[...] = _dot(h, w2_ref[...]) + b2_ref[...]


def _tc_node_final(y, sums, w1, b1, w2, b2):
    return pl.pallas_call(
        _tc_node_final_body,
        grid=(NB,),
        in_specs=[
            pl.BlockSpec((RN, DN), lambda g: (g, 0)),
            pl.BlockSpec((NB, 2, DN), lambda g: (0, 0, 0)),
            pl.BlockSpec((DN, 256), lambda g: (0, 0)),
            pl.BlockSpec((1, 256), lambda g: (0, 0)),
            pl.BlockSpec((256, DN), lambda g: (0, 0)),
            pl.BlockSpec((1, DN), lambda g: (0, 0)),
        ],
        out_specs=pl.BlockSpec((RN, DN), lambda g: (g, 0)),
        out_shape=jax.ShapeDtypeStruct((N, DN), jnp.float32),
    )(y, sums, w1, b1, w2, b2)


def kernel(nfeats, efeats, edge_index, We0, be0, Wn0, bn0, We1, be1, Wn1, bn1,
           Wmn1, bmn1, Wmn2, bmn2, Wme1, bme1, Wme2, bme2):
    pad = I_PAD - E
    src_r = jnp.concatenate(
        [edge_index[0], jnp.full((pad,), N, jnp.int32)]).reshape(NW, K, CHUNK)
    dst_r = jnp.concatenate(
        [edge_index[1], jnp.full((pad,), N, jnp.int32)]).reshape(NW, K, CHUNK)
    ef_r = efeats.reshape(EV, 128)
    zeros_n = jnp.zeros((N_PAD, DE), jnp.float32)

    i8 = jnp.eye(8, dtype=jnp.float32)
    w_blk0 = jnp.kron(i8, We0[2 * DN:])
    w_blk1 = jnp.kron(i8, We1[2 * DN:])
    w_me1k = jnp.kron(i8, Wme1)
    w_me2k = jnp.kron(i8, Wme2)
    b_me1t = jnp.tile(bme1, 8).reshape(1, 256)
    b_me2t = jnp.tile(bme2, 8).reshape(1, 128)
    w_sd0 = jnp.concatenate([We0[:DN], We0[DN:2 * DN]], axis=1)
    w_sd1 = jnp.concatenate([We1[:DN], We1[DN:2 * DN]], axis=1)
    nf_p = jnp.concatenate(
        [nfeats, jnp.zeros((N_PAD - N, DN), jnp.float32)], axis=0)

    ps0, pd0 = _tc_ptables(nf_p, w_sd0)
    q0_r = _tc_edge_pre(ef_r, w_blk0)
    x0, sums0 = _sc_gather(ps0, pd0, q0_r, src_r, dst_r)
    e0_r, q1_r = _tc_edge_act0(x0, sums0.reshape(NW, 32), w_blk1)
    agg0 = _sc_scatter(e0_r, dst_r, zeros_n)
    y0, ns0 = _tc_node_mm(nf_p, agg0, Wn0[:DN], Wn0[DN:])
    nf1_p, ps1, pd1 = _tc_node_apply(y0, ns0, w_sd1)

    x1, sums1 = _sc_gather(ps1, pd1, q1_r, src_r, dst_r)
    e1_r, ef_out_r = _tc_edge_act1(x1, sums1.reshape(NW, 32),
                                   w_me1k, b_me1t, w_me2k, b_me2t)
    agg1 = _sc_scatter(e1_r, dst_r, zeros_n)

    y1, ns1 = _tc_node_mm(nf1_p, agg1, Wn1[:DN], Wn1[DN:])
    nf_out = _tc_node_final(y1, ns1, Wmn1, bmn1.reshape(1, -1),
                            Wmn2, bmn2.reshape(1, -1))
    ef_out = ef_out_r.reshape(E, DE)
    return nf_out, ef_out


# speedup vs baseline: 5.8894x; 1.2878x over previous
import functools

import jax
import jax.numpy as jnp
from jax import lax
from jax.experimental import pallas as pl
from jax.experimental.pallas import tpu as pltpu
from jax.experimental.pallas import tpu_sc as plsc

N = 10000
E = 320000
DN = 128
DE = 16

NC = 2
NS = 16
NW = NC * NS
CHUNK = 128
K = 80
TPE = K * CHUNK
I_PAD = NW * TPE
EV = E // 8
CV = CHUNK // 8
N_PAD = 10240
NB = 4
RN = N_PAD // NB
RN8 = RN // 8
RB = 400
GE = EV // RB
EPS = 1e-5

_HIGH = jax.lax.Precision.HIGHEST


def _sc_gather_body(ps_h, pd_h, q_h, src_h, dst_h, x_h, sums_h,
                    src_v, dst_v, gs0, gs1, gd0, gd1, qx0, qx1, xv0, xv1,
                    sums_v, ss0, ss1, sd0, sd1, sq0, sq1, so0, so1):
    cid = lax.axis_index("c")
    sid = lax.axis_index("s")
    wid = sid * NC + cid
    pltpu.sync_copy(src_h.at[wid], src_v)
    pltpu.sync_copy(dst_h.at[wid], dst_v)
    vrow0 = wid * (TPE // 8)
    vc = jnp.minimum(E - wid * TPE, TPE) // CHUNK
    bufs = ((gs0, gd0, qx0, xv0, ss0, sd0, sq0, so0),
            (gs1, gd1, qx1, xv1, ss1, sd1, sq1, so1))

    def fire(j, b):
        gs, gd, qx = bufs[b][0], bufs[b][1], bufs[b][2]
        ss, sd, sq = bufs[b][4], bufs[b][5], bufs[b][6]
        pltpu.async_copy(ps_h.at[src_v.at[j]], gs, ss)
        pltpu.async_copy(pd_h.at[dst_v.at[j]], gd, sd)
        pltpu.async_copy(q_h.at[pl.ds(vrow0 + j * CV, CV)], qx, sq)

    def wait_in(j, b):
        gs, gd, qx = bufs[b][0], bufs[b][1], bufs[b][2]
        ss, sd, sq = bufs[b][4], bufs[b][5], bufs[b][6]
        pltpu.make_async_copy(ps_h.at[src_v.at[j]], gs, ss).wait()
        pltpu.make_async_copy(pd_h.at[dst_v.at[j]], gd, sd).wait()
        pltpu.make_async_copy(q_h.at[pl.ds(vrow0 + j * CV, CV)], qx, sq).wait()

    fire(0, 0)
    fire(1, 1)
    zv = jnp.zeros((DE,), jnp.float32)
    carry0 = (zv,) * 16

    def pair(t, carry):
        for b in (0, 1):
            gs, gd, qx, xv, _, _, _, so = bufs[b]
            j = 2 * t + b
            wait_in(j, b)

            @pl.when(t > 0)
            def _():
                pltpu.make_async_copy(
                    xv, x_h.at[pl.ds(vrow0 + (j - 2) * CV, CV)], so).wait()

            def rowblk(r, c):
                c = list(c)
                for m in range(8):
                    i = 8 * r + m
                    x = gs[i] + gd[i] + qx[r, pl.ds(16 * m, 16)]
                    xv[r, pl.ds(16 * m, 16)] = x
                    c[m] = c[m] + x
                    c[8 + m] = c[8 + m] + x * x
                return tuple(c)

            carry = lax.fori_loop(0, CV, rowblk, carry)
            pltpu.async_copy(xv, x_h.at[pl.ds(vrow0 + j * CV, CV)], so)

            @pl.when(j + 2 < vc)
            def _():
                fire(j + 2, b)
        return carry

    carry = lax.fori_loop(0, vc // 2, pair, carry0)
    for b in (0, 1):
        xv, so = bufs[b][3], bufs[b][7]
        j_last = vc - 2 + b
        pltpu.make_async_copy(
            xv, x_h.at[pl.ds(vrow0 + j_last * CV, CV)], so).wait()
    s = carry[0]
    s2 = carry[8]
    for m in range(1, 8):
        s = s + carry[m]
        s2 = s2 + carry[8 + m]
    sums_v[0] = s
    sums_v[1] = s2
    pltpu.sync_copy(sums_v, sums_h.at[wid])


def _sc_scatter_body(e_h, dst_h, z_h, agg_h, dst_v, ev0, ev1, et0, et1, acc,
                     sl0, sl1, sw0, sw1):
    cid = lax.axis_index("c")
    sid = lax.axis_index("s")
    wid = sid * NC + cid

    @pl.when(sid == 0)
    def _():
        pltpu.sync_copy(z_h, acc)

    plsc.subcore_barrier()
    pltpu.sync_copy(dst_h.at[wid], dst_v)
    vrow0 = wid * (TPE // 8)
    vc = jnp.minimum(E - wid * TPE, TPE) // CHUNK
    bufs = ((ev0, et0, sl0, sw0), (ev1, et1, sl1, sw1))

    def fire(j, b):
        ev, _, sl, _ = bufs[b]
        pltpu.async_copy(e_h.at[pl.ds(vrow0 + j * CV, CV)], ev, sl)

    fire(0, 0)
    fire(1, 1)

    def pair(t, _):
        for b in (0, 1):
            ev, et, sl, sw = bufs[b]
            j = 2 * t + b
            pltpu.make_async_copy(
                e_h.at[pl.ds(vrow0 + j * CV, CV)], ev, sl).wait()

            @pl.when(t > 0)
            def _():
                pltpu.make_async_copy(et, acc.at[dst_v.at[j - 2]], sw).wait()

            def rowblk(r, c):
                for m in range(8):
                    et[8 * r + m] = ev[r, pl.ds(16 * m, 16)]
                return c

            lax.fori_loop(0, CV, rowblk, 0)
            pltpu.async_copy(et, acc.at[dst_v.at[j]], sw, add=True)

            @pl.when(j + 2 < vc)
            def _():
                fire(j + 2, b)
        return 0

    lax.fori_loop(0, vc // 2, pair, 0)
    for b in (0, 1):
        _, et, _, sw = bufs[b]
        j_last = vc - 2 + b
        pltpu.make_async_copy(et, acc.at[dst_v.at[j_last]], sw).wait()
    plsc.subcore_barrier()
    rp = N_PAD // NS
    pltpu.sync_copy(acc.at[pl.ds(sid * rp, rp)],
                    agg_h.at[cid, pl.ds(sid * rp, rp)])


@functools.cache
def _sc_kernels():
    mesh = plsc.VectorSubcoreMesh(core_axis_name="c", subcore_axis_name="s")
    params = pltpu.CompilerParams(use_tc_tiling_on_sc=False)
    gather = pl.kernel(
        _sc_gather_body,
        compiler_params=params,
        out_type=(
            jax.ShapeDtypeStruct((EV, 128), jnp.float32),
            jax.ShapeDtypeStruct((NW, 2, DE), jnp.float32),
        ),
        mesh=mesh,
        scratch_types=[
            pltpu.VMEM((K, CHUNK), jnp.int32),
            pltpu.VMEM((K, CHUNK), jnp.int32),
            pltpu.VMEM((CHUNK, DE), jnp.float32),
            pltpu.VMEM((CHUNK, DE), jnp.float32),
            pltpu.VMEM((CHUNK, DE), jnp.float32),
            pltpu.VMEM((CHUNK, DE), jnp.float32),
            pltpu.VMEM((CV, 128), jnp.float32),
            pltpu.VMEM((CV, 128), jnp.float32),
            pltpu.VMEM((CV, 128), jnp.float32),
            pltpu.VMEM((CV, 128), jnp.float32),
            pltpu.VMEM((2, DE), jnp.float32),
        ] + [pltpu.SemaphoreType.DMA] * 8,
    )
    scatter = pl.kernel(
        _sc_scatter_body,
        compiler_params=params,
        out_type=jax.ShapeDtypeStruct((NC, N_PAD, DE), jnp.float32),
        mesh=mesh,
        scratch_types=[
            pltpu.VMEM((K, CHUNK), jnp.int32),
            pltpu.VMEM((CV, 128), jnp.float32),
            pltpu.VMEM((CV, 128), jnp.float32),
            pltpu.VMEM((CHUNK, DE), jnp.float32),
            pltpu.VMEM((CHUNK, DE), jnp.float32),
            pltpu.VMEM_SHARED((N_PAD, DE), jnp.float32),
        ] + [pltpu.SemaphoreType.DMA] * 4,
    )
    return gather, scatter


def _sc_gather(ps, pd, q, src_r, dst_r):
    return _sc_kernels()[0](ps, pd, q, src_r, dst_r)


def _sc_scatter(e, dst_r, zeros_n):
    return _sc_kernels()[1](e, dst_r, zeros_n)


def _dot(a, b):
    return jnp.dot(a, b, preferred_element_type=jnp.float32, precision=_HIGH)


_SIN_C = (0.9999999989316655, -0.1666666570294966, 0.008333318542874642,
          -0.0001984041132000804, 2.7533385721326777e-06,
          -2.4706093551925616e-08, 1.353477561276089e-10)


def _fast_sin(x):
    n = jnp.round(x * 0.15915494309189535)
    r = x - n * 6.28125 - n * 0.0019353071795864769
    u = r * r
    p = jnp.full_like(u, _SIN_C[6])
    for k in (5, 4, 3, 2, 1, 0):
        p = p * u + _SIN_C[k]
    return r * p


def _tc_ptables_body(nf8_ref, ws_ref, wd_ref, ps_ref, pd_ref):
    ps_ref[...] = _dot(nf8_ref[...], ws_ref[...])
    pd_ref[...] = _dot(nf8_ref[...], wd_ref[...])


def _tc_ptables(nf8, w_s8, w_d8):
    return pl.pallas_call(
        _tc_ptables_body,
        grid=(NB,),
        in_specs=[
            pl.BlockSpec((RN8, 8 * DN), lambda g: (g, 0)),
            pl.BlockSpec((8 * DN, 128), lambda g: (0, 0)),
            pl.BlockSpec((8 * DN, 128), lambda g: (0, 0)),
        ],
        out_specs=(
            pl.BlockSpec((RN8, 128), lambda g: (g, 0)),
            pl.BlockSpec((RN8, 128), lambda g: (g, 0)),
        ),
        out_shape=(
            jax.ShapeDtypeStruct((N_PAD // 8, 128), jnp.float32),
            jax.ShapeDtypeStruct((N_PAD // 8, 128), jnp.float32),
        ),
    )(nf8, w_s8, w_d8)


def _tc_edge_pre_body(ef_ref, w_ref, q_ref):
    q_ref[...] = _dot(ef_ref[...], w_ref[...])


def _tc_edge_pre(ef_r, w_blk):
    return pl.pallas_call(
        _tc_edge_pre_body,
        grid=(GE,),
        in_specs=[
            pl.BlockSpec((RB, 128), lambda g: (g, 0)),
            pl.BlockSpec((128, 128), lambda g: (0, 0)),
        ],
        out_specs=pl.BlockSpec((RB, 128), lambda g: (g, 0)),
        out_shape=jax.ShapeDtypeStruct((EV, 128), jnp.float32),
    )(ef_r, w_blk)


def _bn_sin_block(x, sums):
    t = jnp.sum(sums, axis=0)
    mu = t[:DE] / E
    var = t[DE:] / E - mu * mu
    inv = lax.rsqrt(var + EPS)
    mu128 = jnp.tile(mu, 8)
    inv128 = jnp.tile(inv, 8)
    return _fast_sin((x - mu128[None, :]) * inv128[None, :])


def _tc_edge_act0_body(x_ref, sums_ref, w_ref, e_ref, q_ref):
    e = _bn_sin_block(x_ref[...], sums_ref[...])
    e_ref[...] = e
    q_ref[...] = _dot(e, w_ref[...])


def _tc_edge_act0(x_r, sums, w_blk_next):
    return pl.pallas_call(
        _tc_edge_act0_body,
        grid=(GE,),
        in_specs=[
            pl.BlockSpec((RB, 128), lambda g: (g, 0)),
            pl.BlockSpec((NW, 32), lambda g: (0, 0)),
            pl.BlockSpec((128, 128), lambda g: (0, 0)),
        ],
        out_specs=(
            pl.BlockSpec((RB, 128), lambda g: (g, 0)),
            pl.BlockSpec((RB, 128), lambda g: (g, 0)),
        ),
        out_shape=(
            jax.ShapeDtypeStruct((EV, 128), jnp.float32),
            jax.ShapeDtypeStruct((EV, 128), jnp.float32),
        ),
    )(x_r, sums, w_blk_next)


def _tc_edge_act1_body(x_ref, sums_ref, w1_ref, b1_ref, w2_ref, b2_ref,
                       e_ref, ef_ref):
    e = _bn_sin_block(x_ref[...], sums_ref[...])
    e_ref[...] = e
    h = _fast_sin(_dot(e, w1_ref[...]) + b1_ref[...])
    ef_ref[...] = _dot(h, w2_ref[...]) + b2_ref[...]


def _tc_edge_act1(x_r, sums, w1k, b1t, w2k, b2t):
    return pl.pallas_call(
        _tc_edge_act1_body,
        grid=(GE,),
        in_specs=[
            pl.BlockSpec((RB, 128), lambda g: (g, 0)),
            pl.BlockSpec((NW, 32), lambda g: (0, 0)),
            pl.BlockSpec((128, 256), lambda g: (0, 0)),
            pl.BlockSpec((1, 256), lambda g: (0, 0)),
            pl.BlockSpec((256, 128), lambda g: (0, 0)),
            pl.BlockSpec((1, 128), lambda g: (0, 0)),
        ],
        out_specs=(
            pl.BlockSpec((RB, 128), lambda g: (g, 0)),
            pl.BlockSpec((RB, 128), lambda g: (g, 0)),
        ),
        out_shape=(
            jax.ShapeDtypeStruct((EV, 128), jnp.float32),
            jax.ShapeDtypeStruct((EV, 128), jnp.float32),
        ),
    )(x_r, sums, w1k, b1t, w2k, b2t)


def _tc_node_mm_body(nf_ref, agg_ref, wnn_ref, wna_ref, y_ref, sums_ref):
    g = pl.program_id(0)
    agg = agg_ref[0] + agg_ref[1]
    y = _dot(nf_ref[...], wnn_ref[...]) + _dot(agg, wna_ref[...])
    rid = g * RN + lax.broadcasted_iota(jnp.int32, (RN, 1), 0)
    y = jnp.where(rid < N, y, 0.0)
    y_ref[...] = y
    sums_ref[0, 0] = jnp.sum(y, axis=0)
    sums_ref[0, 1] = jnp.sum(y * y, axis=0)


def _tc_node_mm(nf_p, agg, wnn, wna):
    return pl.pallas_call(
        _tc_node_mm_body,
        grid=(NB,),
        in_specs=[
            pl.BlockSpec((RN, DN), lambda g: (g, 0)),
            pl.BlockSpec((NC, RN, DE), lambda g: (0, g, 0)),
            pl.BlockSpec((DN, DN), lambda g: (0, 0)),
            pl.BlockSpec((DE, DN), lambda g: (0, 0)),
        ],
        out_specs=(
            pl.BlockSpec((RN, DN), lambda g: (g, 0)),
            pl.BlockSpec((1, 2, DN), lambda g: (g, 0, 0)),
        ),
        out_shape=(
            jax.ShapeDtypeStruct((N_PAD, DN), jnp.float32),
            jax.ShapeDtypeStruct((NB, 2, DN), jnp.float32),
        ),
    )(nf_p, agg, wnn, wna)


def _node_bn_block(y, sums, g):
    t = jnp.sum(sums, axis=0)
    mu = t[0] / N
    var = t[1] / N - mu * mu
    inv = lax.rsqrt(var + EPS)
    act = _fast_sin((y - mu[None, :]) * inv[None, :])
    rid = g * RN + lax.broadcasted_iota(jnp.int32, (RN, 1), 0)
    return jnp.where(rid < N, act, 0.0)


def _tc_node_apply_body(y_ref, sums_ref, nf1_ref):
    nf1_ref[...] = _node_bn_block(y_ref[...], sums_ref[...], pl.program_id(0))


def _tc_node_apply(y, sums):
    return pl.pallas_call(
        _tc_node_apply_body,
        grid=(NB,),
        in_specs=[
            pl.BlockSpec((RN, DN), lambda g: (g, 0)),
            pl.BlockSpec((NB, 2, DN), lambda g: (0, 0, 0)),
        ],
        out_specs=pl.BlockSpec((RN, DN), lambda g: (g, 0)),
        out_shape=jax.ShapeDtypeStruct((N_PAD, DN), jnp.float32),
    )(y, sums)


def _tc_node_final_body(y_ref, sums_ref, w1_ref, b1_ref, w2_ref, b2_ref,
                        out_ref):
    nf2 = _node_bn_block(y_ref[...], sums_ref[...], pl.program_id(0))
    h = _fast_sin(_dot(nf2, w1_ref[...]) + b1_ref[...])
    out_ref[...] = _dot(h, w2_ref[...]) + b2_ref[...]


def _tc_node_final(y, sums, w1, b1, w2, b2):
    return pl.pallas_call(
        _tc_node_final_body,
        grid=(NB,),
        in_specs=[
            pl.BlockSpec((RN, DN), lambda g: (g, 0)),
            pl.BlockSpec((NB, 2, DN), lambda g: (0, 0, 0)),
            pl.BlockSpec((DN, 256), lambda g: (0, 0)),
            pl.BlockSpec((1, 256), lambda g: (0, 0)),
            pl.BlockSpec((256, DN), lambda g: (0, 0)),
            pl.BlockSpec((1, DN), lambda g: (0, 0)),
        ],
        out_specs=pl.BlockSpec((RN, DN), lambda g: (g, 0)),
        out_shape=jax.ShapeDtypeStruct((N, DN), jnp.float32),
    )(y, sums, w1, b1, w2, b2)


def kernel(nfeats, efeats, edge_index, We0, be0, Wn0, bn0, We1, be1, Wn1, bn1,
           Wmn1, bmn1, Wmn2, bmn2, Wme1, bme1, Wme2, bme2):
    pad = I_PAD - E
    src_r = jnp.concatenate(
        [edge_index[0], jnp.full((pad,), N, jnp.int32)]).reshape(NW, K, CHUNK)
    dst_r = jnp.concatenate(
        [edge_index[1], jnp.full((pad,), N, jnp.int32)]).reshape(NW, K, CHUNK)
    ef_r = efeats.reshape(EV, 128)
    zeros_n = jnp.zeros((N_PAD, DE), jnp.float32)

    i8 = jnp.eye(8, dtype=jnp.float32)
    w_blk0 = jnp.kron(i8, We0[2 * DN:])
    w_blk1 = jnp.kron(i8, We1[2 * DN:])
    w_me1k = jnp.kron(i8, Wme1)
    w_me2k = jnp.kron(i8, Wme2)
    b_me1t = jnp.tile(bme1, 8).reshape(1, 256)
    b_me2t = jnp.tile(bme2, 8).reshape(1, 128)
    w_s80 = jnp.kron(i8, We0[:DN])
    w_d80 = jnp.kron(i8, We0[DN:2 * DN])
    w_s81 = jnp.kron(i8, We1[:DN])
    w_d81 = jnp.kron(i8, We1[DN:2 * DN])
    nf_p = jnp.concatenate(
        [nfeats, jnp.zeros((N_PAD - N, DN), jnp.float32)], axis=0)

    ps0, pd0 = _tc_ptables(nf_p.reshape(N_PAD // 8, 8 * DN), w_s80, w_d80)
    q0_r = _tc_edge_pre(ef_r, w_blk0)
    x0, sums0 = _sc_gather(ps0.reshape(N_PAD, DE), pd0.reshape(N_PAD, DE),
                           q0_r, src_r, dst_r)
    e0_r, q1_r = _tc_edge_act0(x0, sums0.reshape(NW, 32), w_blk1)
    agg0 = _sc_scatter(e0_r, dst_r, zeros_n)
    y0, ns0 = _tc_node_mm(nf_p, agg0, Wn0[:DN], Wn0[DN:])
    nf1_p = _tc_node_apply(y0, ns0)
    ps1, pd1 = _tc_ptables(nf1_p.reshape(N_PAD // 8, 8 * DN), w_s81, w_d81)

    x1, sums1 = _sc_gather(ps1.reshape(N_PAD, DE), pd1.reshape(N_PAD, DE),
                           q1_r, src_r, dst_r)
    e1_r, ef_out_r = _tc_edge_act1(x1, sums1.reshape(NW, 32),
                                   w_me1k, b_me1t, w_me2k, b_me2t)
    agg1 = _sc_scatter(e1_r, dst_r, zeros_n)

    y1, ns1 = _tc_node_mm(nf1_p, agg1, Wn1[:DN], Wn1[DN:])
    nf_out = _tc_node_final(y1, ns1, Wmn1, bmn1.reshape(1, -1),
                            Wmn2, bmn2.reshape(1, -1))
    ef_out = ef_out_r.reshape(E, DE)
    return nf_out, ef_out
